# baseline (speedup 1.0000x reference)
import jax
import jax.numpy as jnp
from jax.experimental import pallas as pl
from jax.experimental.pallas import tpu as pltpu


_ROWS = 150528
_LANES = 128
_NCHUNKS = 8
_CHUNK = _ROWS // _NCHUNKS


def _copy_kernel(x_hbm, o_hbm, sems):
    for c in range(_NCHUNKS):
        pltpu.make_async_copy(
            x_hbm.at[pl.ds(c * _CHUNK, _CHUNK), :],
            o_hbm.at[pl.ds(c * _CHUNK, _CHUNK), :],
            sems.at[c],
        ).start()
    for c in range(_NCHUNKS):
        pltpu.make_async_copy(
            x_hbm.at[pl.ds(c * _CHUNK, _CHUNK), :],
            o_hbm.at[pl.ds(c * _CHUNK, _CHUNK), :],
            sems.at[c],
        ).wait()


def kernel(feature_batch, box_batch):
    x = feature_batch.reshape(_ROWS, _LANES)
    out = pl.pallas_call(
        _copy_kernel,
        in_specs=[pl.BlockSpec(memory_space=pltpu.MemorySpace.HBM)],
        out_specs=pl.BlockSpec(memory_space=pltpu.MemorySpace.HBM),
        out_shape=jax.ShapeDtypeStruct((_ROWS, _LANES), jnp.float32),
        scratch_shapes=[pltpu.SemaphoreType.DMA((_NCHUNKS,))],
    )(x)
    return out.reshape(feature_batch.shape)


# speedup vs baseline: 10.0357x; 10.0357x over previous
import jax
import jax.numpy as jnp
from jax.experimental import pallas as pl
from jax.experimental.pallas import tpu as pltpu


_ROWS = 37632
_LANES = 512
_GRID = 14
_BLOCK_ROWS = _ROWS // _GRID


def _copy_kernel(x_ref, o_ref):
    o_ref[...] = x_ref[...]


def kernel(feature_batch, box_batch):
    x = feature_batch.reshape(_ROWS, _LANES)
    out = pl.pallas_call(
        _copy_kernel,
        grid=(_GRID,),
        in_specs=[pl.BlockSpec((_BLOCK_ROWS, _LANES), lambda i: (i, 0))],
        out_specs=pl.BlockSpec((_BLOCK_ROWS, _LANES), lambda i: (i, 0)),
        out_shape=jax.ShapeDtypeStruct((_ROWS, _LANES), jnp.float32),
        compiler_params=pltpu.CompilerParams(
            dimension_semantics=("parallel",),
        ),
    )(x)
    return out.reshape(feature_batch.shape)


# speedup vs baseline: 45.6590x; 4.5497x over previous
import jax
import jax.numpy as jnp
from jax.experimental import pallas as pl
from jax.experimental.pallas import tpu as pltpu


_IMGS = 384
_GRID = 16
_BLOCK = _IMGS // _GRID


def _copy_kernel(x_ref, o_ref):
    o_ref[...] = x_ref[...]


def kernel(feature_batch, box_batch):
    x = feature_batch.reshape(_IMGS, 224, 224)
    out = pl.pallas_call(
        _copy_kernel,
        grid=(_GRID,),
        in_specs=[pl.BlockSpec((_BLOCK, 224, 224), lambda i: (i, 0, 0))],
        out_specs=pl.BlockSpec((_BLOCK, 224, 224), lambda i: (i, 0, 0)),
        out_shape=jax.ShapeDtypeStruct((_IMGS, 224, 224), jnp.float32),
        compiler_params=pltpu.CompilerParams(
            dimension_semantics=("parallel",),
        ),
    )(x)
    return out.reshape(feature_batch.shape)


# speedup vs baseline: 46.7921x; 1.0248x over previous
import jax
import jax.numpy as jnp
from jax.experimental import pallas as pl
from jax.experimental.pallas import tpu as pltpu


_IMGS = 384
_GRID = 8
_BLOCK = _IMGS // _GRID


def _copy_kernel(x_ref, o_ref):
    o_ref[...] = x_ref[...]


def kernel(feature_batch, box_batch):
    x = feature_batch.reshape(_IMGS, 224, 224)
    out = pl.pallas_call(
        _copy_kernel,
        grid=(_GRID,),
        in_specs=[pl.BlockSpec((_BLOCK, 224, 224), lambda i: (i, 0, 0))],
        out_specs=pl.BlockSpec((_BLOCK, 224, 224), lambda i: (i, 0, 0)),
        out_shape=jax.ShapeDtypeStruct((_IMGS, 224, 224), jnp.float32),
        compiler_params=pltpu.CompilerParams(
            dimension_semantics=("parallel",),
        ),
    )(x)
    return out.reshape(feature_batch.shape)


# speedup vs baseline: 46.8957x; 1.0022x over previous
import jax
import jax.numpy as jnp
from jax.experimental import pallas as pl
from jax.experimental.pallas import tpu as pltpu


_IMGS = 384
_GRID = 6
_BLOCK = _IMGS // _GRID


def _copy_kernel(x_ref, o_ref):
    o_ref[...] = x_ref[...]


def kernel(feature_batch, box_batch):
    x = feature_batch.reshape(_IMGS, 224, 224)
    out = pl.pallas_call(
        _copy_kernel,
        grid=(_GRID,),
        in_specs=[pl.BlockSpec((_BLOCK, 224, 224), lambda i: (i, 0, 0))],
        out_specs=pl.BlockSpec((_BLOCK, 224, 224), lambda i: (i, 0, 0)),
        out_shape=jax.ShapeDtypeStruct((_IMGS, 224, 224), jnp.float32),
        compiler_params=pltpu.CompilerParams(
            dimension_semantics=("parallel",),
        ),
    )(x)
    return out.reshape(feature_batch.shape)
